# Initial kernel scaffold; baseline (speedup 1.0000x reference)
import jax
import jax.numpy as jnp
from jax.experimental import pallas as pl


def kernel(queries, memory_bank):
    raise NotImplementedError("write your pallas kernel here")


# speedup vs baseline: 1.0741x; 1.0741x over previous
import jax
import jax.numpy as jnp
from jax.experimental import pallas as pl
from jax.experimental.pallas import tpu as pltpu

B = 8
P = 784
D = 1024
K = 16384
Q = B * P
NN = 9

BQ = 896
BK = 2048


def _min_dist_kernel(q_ref, mt_ref, out_ref):
    nk = pl.num_programs(1)
    j = pl.program_id(1)
    q = q_ref[...]
    mt = mt_ref[...]
    qm = jnp.dot(q, mt, preferred_element_type=jnp.float32)
    m2 = jnp.sum(mt * mt, axis=0)
    part = m2[None, :] - 2.0 * qm
    bmin = jnp.min(part, axis=1)[:, None]

    @pl.when(j == 0)
    def _():
        out_ref[...] = bmin

    @pl.when(j > 0)
    def _():
        out_ref[...] = jnp.minimum(out_ref[...], bmin)

    @pl.when(j == nk - 1)
    def _():
        q2 = jnp.sum(q * q, axis=1)[:, None]
        out_ref[...] = jnp.sqrt(jnp.maximum(out_ref[...] + q2, 1e-12))


def _select_row_kernel(ps_ref, q_ref, mt_ref, d2_ref, sstar_ref, qsel_ref):
    j = pl.program_id(0)

    @pl.when(j == 0)
    def _():
        ps = ps_ref[...]
        sstar_ref[...] = jnp.max(ps, axis=1)[:, None]
        idx = jnp.argmax(ps, axis=1)
        flat = idx + jax.lax.iota(jnp.int32, B) * P
        onehot = (flat[:, None] ==
                  jax.lax.broadcasted_iota(jnp.int32, (B, Q), 1)).astype(jnp.float32)
        qsel_ref[...] = jnp.dot(onehot, q_ref[...],
                                preferred_element_type=jnp.float32)

    qsel = qsel_ref[...]
    mt = mt_ref[...]
    qm = jnp.dot(qsel, mt, preferred_element_type=jnp.float32)
    m2 = jnp.sum(mt * mt, axis=0)[None, :]
    q2 = jnp.sum(qsel * qsel, axis=1)[:, None]
    d2_ref[...] = q2 + m2 - 2.0 * qm


def _rescore_kernel(d2_ref, sstar_ref, out_ref):
    d = jnp.sqrt(jnp.maximum(d2_ref[...], 1e-12))
    col = jax.lax.broadcasted_iota(jnp.int32, (B, K), 1)
    nn = []
    for _ in range(NN):
        cur = jnp.min(d, axis=1)
        nn.append(cur)
        amin = jnp.argmin(d, axis=1)
        d = jnp.where(col == amin[:, None], jnp.inf, d)
    nn_dists = jnp.stack(nn, axis=1)
    sstar = sstar_ref[...][:, 0]
    mx = nn_dists[:, NN - 1]
    weights = 1.0 - jnp.exp(sstar - mx) / jnp.sum(
        jnp.exp(nn_dists - mx[:, None]), axis=1)
    out_ref[...] = (weights * sstar)[:, None]


def kernel(queries, memory_bank):
    mt = memory_bank.T

    patch_flat = pl.pallas_call(
        _min_dist_kernel,
        grid=(Q // BQ, K // BK),
        in_specs=[
            pl.BlockSpec((BQ, D), lambda i, j: (i, 0)),
            pl.BlockSpec((D, BK), lambda i, j: (0, j)),
        ],
        out_specs=pl.BlockSpec((BQ, 1), lambda i, j: (i, 0)),
        out_shape=jax.ShapeDtypeStruct((Q, 1), jnp.float32),
        compiler_params=pltpu.CompilerParams(
            dimension_semantics=("parallel", "arbitrary")),
    )(queries, mt)
    patch_scores = patch_flat[:, 0].reshape(B, P)

    d2_rows, sstar = pl.pallas_call(
        _select_row_kernel,
        grid=(K // BK,),
        in_specs=[
            pl.BlockSpec((B, P), lambda j: (0, 0)),
            pl.BlockSpec((Q, D), lambda j: (0, 0)),
            pl.BlockSpec((D, BK), lambda j: (0, j)),
        ],
        out_specs=[
            pl.BlockSpec((B, BK), lambda j: (0, j)),
            pl.BlockSpec((B, 1), lambda j: (0, 0)),
        ],
        out_shape=[
            jax.ShapeDtypeStruct((B, K), jnp.float32),
            jax.ShapeDtypeStruct((B, 1), jnp.float32),
        ],
        scratch_shapes=[pltpu.VMEM((B, D), jnp.float32)],
        compiler_params=pltpu.CompilerParams(
            dimension_semantics=("arbitrary",)),
    )(patch_scores, queries, mt)

    image_scores = pl.pallas_call(
        _rescore_kernel,
        in_specs=[
            pl.BlockSpec((B, K), lambda: (0, 0)),
            pl.BlockSpec((B, 1), lambda: (0, 0)),
        ],
        out_specs=pl.BlockSpec((B, 1), lambda: (0, 0)),
        out_shape=jax.ShapeDtypeStruct((B, 1), jnp.float32),
    )(d2_rows, sstar)[:, 0]

    return image_scores, patch_scores
